# Initial kernel scaffold; baseline (speedup 1.0000x reference)
import jax
import jax.numpy as jnp
from jax.experimental import pallas as pl


def kernel(x, edge_index, W1, b1, Wmu, bmu, Wls, bls):
    raise NotImplementedError("write your pallas kernel here")


# speedup vs baseline: 15.0246x; 15.0246x over previous
import functools

import jax
import jax.numpy as jnp
from jax import lax
from jax.experimental import pallas as pl
from jax.experimental.pallas import tpu as pltpu
from jax.experimental.pallas import tpu_sc as plsc

N_NODES = 10000
N_EDGES = 320000
CH = 128
OUT_CH = 64

NPAD = 10240
CHUNK = 128
NC = 2
NS = 16
NW = NC * NS
K = 79
EPAD = NW * K * CHUNK
ROWS_PER_TILE = NPAD // NS
BM = 512

_mesh = plsc.VectorSubcoreMesh(core_axis_name="c", subcore_axis_name="s")


@functools.partial(
    pl.kernel,
    out_type=jax.ShapeDtypeStruct((NC, NPAD), jnp.float32),
    mesh=_mesh,
    scratch_types=[
        pltpu.VMEM((K, CHUNK), jnp.int32),
        pltpu.VMEM((CHUNK,), jnp.float32),
        pltpu.VMEM((ROWS_PER_TILE,), jnp.float32),
        pltpu.VMEM_SHARED((NPAD,), jnp.float32),
        pltpu.SemaphoreType.DMA,
    ],
)
def _sc_degree(dstw_hbm, degp_hbm, dst_v, ones_v, zer_v, acc, sem):
    c = lax.axis_index("c")
    s = lax.axis_index("s")
    wid = c * NS + s
    pltpu.sync_copy(dstw_hbm.at[wid], dst_v)

    @pl.loop(0, CHUNK // 16)
    def _fill_ones(j):
        ones_v[pl.ds(j * 16, 16)] = jnp.ones((16,), jnp.float32)

    @pl.loop(0, ROWS_PER_TILE // 16)
    def _fill_zeros(j):
        zer_v[pl.ds(j * 16, 16)] = jnp.zeros((16,), jnp.float32)

    pltpu.sync_copy(zer_v, acc.at[pl.ds(s * ROWS_PER_TILE, ROWS_PER_TILE)])
    plsc.subcore_barrier()

    @pl.loop(0, K)
    def _scatter(j):
        pltpu.sync_copy(ones_v, acc.at[dst_v.at[j]], add=True)

    plsc.subcore_barrier()
    pltpu.sync_copy(acc.at[pl.ds(s * ROWS_PER_TILE, ROWS_PER_TILE)],
                    degp_hbm.at[c, pl.ds(s * ROWS_PER_TILE, ROWS_PER_TILE)])


@functools.partial(
    pl.kernel,
    out_type=jax.ShapeDtypeStruct((NC, NPAD, CH), jnp.float32),
    mesh=_mesh,
    scratch_types=[
        pltpu.VMEM((K, CHUNK), jnp.int32),
        pltpu.VMEM((K, CHUNK), jnp.int32),
        pltpu.VMEM((CHUNK, CH), jnp.float32),
        pltpu.VMEM_SHARED((NPAD, CH), jnp.float32),
        pltpu.SemaphoreType.DMA,
    ],
)
def _sc_prop(z_hbm, srcw_hbm, dstw_hbm, aggp_hbm, src_v, dst_v, rows_v, acc,
             sem):
    c = lax.axis_index("c")
    s = lax.axis_index("s")
    wid = c * NS + s
    pltpu.sync_copy(srcw_hbm.at[wid], src_v)
    pltpu.sync_copy(dstw_hbm.at[wid], dst_v)

    @pl.loop(0, CHUNK)
    def _zr(i):
        @pl.loop(0, CH // 16)
        def _zc(j):
            rows_v[i, pl.ds(j * 16, 16)] = jnp.zeros((16,), jnp.float32)

    @pl.loop(0, ROWS_PER_TILE // CHUNK)
    def _za(k):
        pltpu.sync_copy(
            rows_v, acc.at[pl.ds(s * ROWS_PER_TILE + k * CHUNK, CHUNK)])

    plsc.subcore_barrier()

    @pl.loop(0, K)
    def _edge(j):
        pltpu.async_copy(z_hbm.at[src_v.at[j]], rows_v, sem).wait()
        pltpu.sync_copy(rows_v, acc.at[dst_v.at[j]], add=True)

    plsc.subcore_barrier()

    @pl.loop(0, ROWS_PER_TILE // CHUNK)
    def _out(k):
        r0 = s * ROWS_PER_TILE + k * CHUNK
        pltpu.sync_copy(acc.at[pl.ds(r0, CHUNK)],
                        aggp_hbm.at[c, pl.ds(r0, CHUNK)])


def _deg_dinv(degp_blk):
    deg = degp_blk[0, :] + degp_blk[1, :] + 1.0
    return lax.rsqrt(deg)[:, None], (1.0 / deg)[:, None]


def _tc_scale_mm(x_ref, w_ref, degp_ref, zs_ref, self_ref):
    xw = jnp.dot(x_ref[...], w_ref[...], preferred_element_type=jnp.float32)
    dinv, dinv2 = _deg_dinv(degp_ref[...])
    zs_ref[...] = xw * dinv
    self_ref[...] = xw * dinv2


def _tc_mid(aggp_ref, self1_ref, degp_ref, b1_ref, wcat_ref, zs2_ref,
            self2_ref):
    dinv, dinv2 = _deg_dinv(degp_ref[...])
    agg = aggp_ref[0] + aggp_ref[1]
    h = jnp.maximum(agg * dinv + self1_ref[...] + b1_ref[...], 0.0)
    hw = jnp.dot(h, wcat_ref[...], preferred_element_type=jnp.float32)
    zs2_ref[...] = hw * dinv
    self2_ref[...] = hw * dinv2


def _tc_final(aggp_ref, self2_ref, degp_ref, bcat_ref, out_ref):
    dinv, _ = _deg_dinv(degp_ref[...])
    agg = aggp_ref[0] + aggp_ref[1]
    out_ref[...] = agg * dinv + self2_ref[...] + bcat_ref[...]


def _row_specs(n_extra):
    return [pl.BlockSpec((BM, CH), lambda i: (i, 0)) for _ in range(n_extra)]


_spec_rows = pl.BlockSpec((BM, CH), lambda i: (i, 0))
_spec_w = pl.BlockSpec((CH, CH), lambda i: (0, 0))
_spec_degp = pl.BlockSpec((NC, BM), lambda i: (0, i))
_spec_aggp = pl.BlockSpec((NC, BM, CH), lambda i: (0, i, 0))
_spec_b = pl.BlockSpec((1, CH), lambda i: (0, 0))
_GRID = (NPAD // BM,)


def kernel(x, edge_index, W1, b1, Wmu, bmu, Wls, bls):
    f32 = jnp.float32
    xpad = jnp.pad(x.astype(f32), ((0, NPAD - N_NODES), (0, 0)))
    src = edge_index[0].astype(jnp.int32)
    dst = edge_index[1].astype(jnp.int32)
    pad_e = EPAD - N_EDGES
    fill = jnp.full((pad_e,), N_NODES, jnp.int32)
    srcw = jnp.concatenate([src, fill]).reshape(NW, K, CHUNK)
    dstw = jnp.concatenate([dst, fill]).reshape(NW, K, CHUNK)

    wcat = jnp.concatenate([Wmu, Wls], axis=1).astype(f32)
    bcat = jnp.concatenate([bmu, bls]).reshape(1, CH).astype(f32)
    b1r = b1.reshape(1, CH).astype(f32)

    degp = _sc_degree(dstw)

    zs1, self1 = pl.pallas_call(
        _tc_scale_mm,
        grid=_GRID,
        in_specs=[_spec_rows, _spec_w, _spec_degp],
        out_specs=[_spec_rows, _spec_rows],
        out_shape=[jax.ShapeDtypeStruct((NPAD, CH), f32)] * 2,
    )(xpad, W1.astype(f32), degp)

    aggp1 = _sc_prop(zs1, srcw, dstw)

    zs2, self2 = pl.pallas_call(
        _tc_mid,
        grid=_GRID,
        in_specs=[_spec_aggp, _spec_rows, _spec_degp, _spec_b, _spec_w],
        out_specs=[_spec_rows, _spec_rows],
        out_shape=[jax.ShapeDtypeStruct((NPAD, CH), f32)] * 2,
    )(aggp1, self1, degp, b1r, wcat)

    aggp2 = _sc_prop(zs2, srcw, dstw)

    out = pl.pallas_call(
        _tc_final,
        grid=_GRID,
        in_specs=[_spec_aggp, _spec_rows, _spec_degp, _spec_b],
        out_specs=_spec_rows,
        out_shape=jax.ShapeDtypeStruct((NPAD, CH), f32),
    )(aggp2, self2, degp, bcat)

    mu = out[:N_NODES, :OUT_CH]
    logstd = out[:N_NODES, OUT_CH:]
    return (mu, logstd)
